# Initial kernel scaffold; baseline (speedup 1.0000x reference)
import jax
import jax.numpy as jnp
from jax.experimental import pallas as pl


def kernel(context_ids, doc_ids, target_noise_ids, D, W, O):
    raise NotImplementedError("write your pallas kernel here")


# speedup vs baseline: 1.1269x; 1.1269x over previous
import jax
import jax.numpy as jnp
from jax import lax
from jax.experimental import pallas as pl
from jax.experimental.pallas import tpu as pltpu
from jax.experimental.pallas import tpu_sc as plsc

B, C, NP1 = 4096, 20, 26
ND, NW, V = 100, 10000, 64
L = 16
NC, NS = 2, 16
NWORK = NC * NS
BPW = B // NWORK
CB = 32
NCHUNK = BPW // CB
IW = 128
NJD = CB * C // IW
NJO = (CB * NP1 + IW - 1) // IW


def _sc_gather_kernel(ctx_hbm, fidx_hbm, tn_hbm, dflat_hbm, w_hbm, ot_hbm,
                      x_hbm, otg_hbm,
                      ctx_v, tn_v, didx_v, tgt_v, dbuf, wbuf,
                      zbuf, zidx_v, xacc, semd, semw, semo):
    cid = lax.axis_index("c")
    sid = lax.axis_index("s")
    wid = cid * NS + sid
    iota = lax.iota(jnp.int32, L)

    zero = jnp.zeros((L,), jnp.float32)
    for r in range(BPW):
        for s in range(V // L):
            zbuf[r, pl.ds(s * L, L)] = zero
    for i in range(BPW // L):
        zidx_v[0, pl.ds(i * L, L)] = iota + (sid * BPW + i * L)
    pltpu.sync_copy(zbuf, xacc.at[zidx_v.at[0]])

    for k in range(NCHUNK):
        b0 = (wid * BPW + k * CB).astype(jnp.int32)
        gchunk = wid * NCHUNK + k
        pltpu.sync_copy(fidx_hbm.at[pl.ds(b0 * C // IW, NJD)], didx_v)
        pltpu.sync_copy(ctx_hbm.at[pl.ds(b0 * C // IW, NJD)], ctx_v)
        pltpu.sync_copy(tn_hbm.at[pl.ds(gchunk * NJO, NJO)], tn_v)

        base = sid * BPW + k * CB
        for i in range(CB * C // L):
            j, col = i * L // IW, i * L % IW
            bl = lax.div(iota + (i * L), C)
            tgt_v[j, pl.ds(col, L)] = bl + base

        gds = [pltpu.async_copy(dflat_hbm.at[didx_v.at[j]],
                                dbuf.at[pl.ds(j * IW, IW)], semd)
               for j in range(NJD)]
        gws = [pltpu.async_copy(w_hbm.at[ctx_v.at[j]],
                                wbuf.at[pl.ds(j * IW, IW)], semw)
               for j in range(NJD)]

        for g in gds:
            g.wait()
        for j in range(NJD):
            pltpu.sync_copy(dbuf.at[pl.ds(j * IW, IW)],
                            xacc.at[tgt_v.at[j]], add=True)
        for g in gws:
            g.wait()
        for j in range(NJD):
            pltpu.sync_copy(wbuf.at[pl.ds(j * IW, IW)],
                            xacc.at[tgt_v.at[j]], add=True)

        gos = [pltpu.async_copy(ot_hbm.at[tn_v.at[j]],
                                dbuf.at[pl.ds(j * IW, IW)], semo)
               for j in range(NJO)]
        for g in gos:
            g.wait()
        pltpu.sync_copy(dbuf.at[pl.ds(0, CB * NP1)],
                        otg_hbm.at[pl.ds(b0 * NP1, CB * NP1)])

    plsc.subcore_barrier()
    pltpu.sync_copy(xacc.at[pl.ds(sid * BPW, BPW)],
                    x_hbm.at[pl.ds(wid * BPW, BPW)])


def _tc_dot_kernel(x_ref, og_ref, out_ref):
    x = x_ref[...]
    og = og_ref[...]
    out_ref[...] = jnp.sum(og * x[:, None, :], axis=-1)


def _run_sc(ctx2d, fidx2d, tn2d, dflat, W, ot):
    mesh = plsc.VectorSubcoreMesh(core_axis_name="c", subcore_axis_name="s")
    sc = pl.kernel(
        _sc_gather_kernel,
        out_type=(
            jax.ShapeDtypeStruct((B, V), jnp.float32),
            jax.ShapeDtypeStruct((B * NP1, V), jnp.float32),
        ),
        mesh=mesh,
        compiler_params=pltpu.CompilerParams(
            needs_layout_passes=False, use_tc_tiling_on_sc=False),
        scratch_types=[
            pltpu.VMEM((NJD, IW), jnp.int32),
            pltpu.VMEM((NJO, IW), jnp.int32),
            pltpu.VMEM((NJD, IW), jnp.int32),
            pltpu.VMEM((NJD, IW), jnp.int32),
            pltpu.VMEM((NJO * IW, V), jnp.float32),
            pltpu.VMEM((CB * C, V), jnp.float32),
            pltpu.VMEM((BPW, V), jnp.float32),
            pltpu.VMEM((1, IW), jnp.int32),
            pltpu.VMEM_SHARED((NS * BPW, V), jnp.float32),
            pltpu.SemaphoreType.DMA,
            pltpu.SemaphoreType.DMA,
            pltpu.SemaphoreType.DMA,
        ],
    )
    return sc(ctx2d, fidx2d, tn2d, dflat, W, ot)


def kernel(context_ids, doc_ids, target_noise_ids, D, W, O):
    ctx = context_ids.astype(jnp.int32)
    doc = doc_ids.astype(jnp.int32)
    ctx2d = ctx.reshape(B * C // IW, IW)
    fidx2d = (doc[:, None] * NW + ctx).reshape(B * C // IW, IW)
    tn = target_noise_ids.astype(jnp.int32).reshape(B // CB, CB * NP1)
    tn2d = jnp.pad(tn, ((0, 0), (0, NJO * IW - CB * NP1))).reshape(
        B // CB * NJO, IW)
    dflat = D.reshape(ND * NW, V)
    ot = O.T
    x, otg = _run_sc(ctx2d, fidx2d, tn2d, dflat, W, ot)

    BB = 256
    out = pl.pallas_call(
        _tc_dot_kernel,
        grid=(B // BB,),
        in_specs=[
            pl.BlockSpec((BB, V), lambda i: (i, 0)),
            pl.BlockSpec((BB, NP1, V), lambda i: (i, 0, 0)),
        ],
        out_specs=pl.BlockSpec((BB, NP1), lambda i: (i, 0)),
        out_shape=jax.ShapeDtypeStruct((B, NP1), jnp.float32),
    )(x, otg.reshape(B, NP1, V))
    return out


# speedup vs baseline: 1.4071x; 1.2486x over previous
import jax
import jax.numpy as jnp
from jax import lax
from jax.experimental import pallas as pl
from jax.experimental.pallas import tpu as pltpu
from jax.experimental.pallas import tpu_sc as plsc

B, C, NP1 = 4096, 20, 26
ND, NW, V = 100, 10000, 64
L = 16
NC, NS = 2, 16
NWORK = NC * NS
BPW = B // NWORK
IW = 128
NJD = BPW * C // IW
NJO = BPW * NP1 // IW
NB = 3


def _sc_gather_kernel(ctx_hbm, fidx_hbm, tn_hbm, dflat_hbm, w_hbm, ot_hbm,
                      x_hbm, otg_hbm,
                      ctx_v, tn_v, didx_v, tgt_v, dbuf, wbuf, obuf,
                      zbuf, zidx_v, xacc,
                      semdg, semda, semwg, semwa, semog, semow):
    cid = lax.axis_index("c")
    sid = lax.axis_index("s")
    wid = cid * NS + sid
    iota = lax.iota(jnp.int32, L)
    base = sid * BPW

    zero = jnp.zeros((L,), jnp.float32)
    for r in range(IW):
        for s in range(V // L):
            zbuf[r, pl.ds(s * L, L)] = zero
    for i in range(IW // L):
        zidx_v[0, pl.ds(i * L, L)] = iota + (base + i * L)
    zd = pltpu.async_copy(zbuf, xacc.at[zidx_v.at[0]], semda)

    pltpu.sync_copy(fidx_hbm.at[pl.ds(wid * NJD, NJD)], didx_v)
    pltpu.sync_copy(ctx_hbm.at[pl.ds(wid * NJD, NJD)], ctx_v)
    pltpu.sync_copy(tn_hbm.at[pl.ds(wid * NJO, NJO)], tn_v)
    for i in range(BPW * C // L):
        j, col = i * L // IW, i * L % IW
        bl = lax.div(iota + (i * L), C)
        tgt_v[j, pl.ds(col, L)] = bl + base
    zd.wait()

    dg = [None] * NJD
    da = [None] * NJD
    wg = [None] * NJD
    wa = [None] * NJD
    og = [None] * NJO
    ow = [None] * NJO
    for t in range(NJO + 1):
        if t < NJD:
            if t >= NB:
                da[t - NB].wait()
            dg[t] = pltpu.async_copy(dflat_hbm.at[didx_v.at[t]],
                                     dbuf.at[t % NB], semdg)
            if t >= NB:
                wa[t - NB].wait()
            wg[t] = pltpu.async_copy(w_hbm.at[ctx_v.at[t]],
                                     wbuf.at[t % NB], semwg)
        if t < NJO:
            if t >= NB:
                ow[t - NB].wait()
            og[t] = pltpu.async_copy(ot_hbm.at[tn_v.at[t]],
                                     obuf.at[t % NB], semog)
        u = t - 1
        if 0 <= u < NJD:
            dg[u].wait()
            da[u] = pltpu.async_copy(dbuf.at[u % NB], xacc.at[tgt_v.at[u]],
                                     semda, add=True)
            wg[u].wait()
            wa[u] = pltpu.async_copy(wbuf.at[u % NB], xacc.at[tgt_v.at[u]],
                                     semwa, add=True)
        if 0 <= u < NJO:
            og[u].wait()
            ow[u] = pltpu.async_copy(
                obuf.at[u % NB],
                otg_hbm.at[pl.ds(wid * (BPW * NP1) + u * IW, IW)], semow)

    for u in range(max(NJD - NB, 0), NJD):
        da[u].wait()
        wa[u].wait()
    for u in range(max(NJO - NB, 0), NJO):
        ow[u].wait()

    plsc.subcore_barrier()
    pltpu.sync_copy(xacc.at[pl.ds(base, BPW)],
                    x_hbm.at[pl.ds(wid * BPW, BPW)])


def _tc_transpose_kernel(o_ref, ot_ref):
    ot_ref[...] = o_ref[...].T


def _tc_dot_kernel(x_ref, og_ref, out_ref):
    x = x_ref[...]
    og = og_ref[...]
    out_ref[...] = jnp.sum(og * x[:, None, :], axis=-1)


def _run_sc(ctx2d, fidx2d, tn2d, dflat, W, ot):
    mesh = plsc.VectorSubcoreMesh(core_axis_name="c", subcore_axis_name="s")
    sc = pl.kernel(
        _sc_gather_kernel,
        out_type=(
            jax.ShapeDtypeStruct((B, V), jnp.float32),
            jax.ShapeDtypeStruct((B * NP1, V), jnp.float32),
        ),
        mesh=mesh,
        compiler_params=pltpu.CompilerParams(
            needs_layout_passes=False, use_tc_tiling_on_sc=False),
        scratch_types=[
            pltpu.VMEM((NJD, IW), jnp.int32),
            pltpu.VMEM((NJO, IW), jnp.int32),
            pltpu.VMEM((NJD, IW), jnp.int32),
            pltpu.VMEM((NJD, IW), jnp.int32),
            pltpu.VMEM((NB, IW, V), jnp.float32),
            pltpu.VMEM((NB, IW, V), jnp.float32),
            pltpu.VMEM((NB, IW, V), jnp.float32),
            pltpu.VMEM((IW, V), jnp.float32),
            pltpu.VMEM((1, IW), jnp.int32),
            pltpu.VMEM_SHARED((NS * BPW, V), jnp.float32),
            pltpu.SemaphoreType.DMA,
            pltpu.SemaphoreType.DMA,
            pltpu.SemaphoreType.DMA,
            pltpu.SemaphoreType.DMA,
            pltpu.SemaphoreType.DMA,
            pltpu.SemaphoreType.DMA,
        ],
    )
    return sc(ctx2d, fidx2d, tn2d, dflat, W, ot)


def kernel(context_ids, doc_ids, target_noise_ids, D, W, O):
    ctx = context_ids.astype(jnp.int32)
    doc = doc_ids.astype(jnp.int32)
    ctx2d = ctx.reshape(B * C // IW, IW)
    fidx2d = (doc[:, None] * NW + ctx).reshape(B * C // IW, IW)
    tn2d = target_noise_ids.astype(jnp.int32).reshape(B * NP1 // IW, IW)
    dflat = D.reshape(ND * NW, V)

    ot = pl.pallas_call(
        _tc_transpose_kernel,
        out_shape=jax.ShapeDtypeStruct((NW, V), jnp.float32),
    )(O)

    x, otg = _run_sc(ctx2d, fidx2d, tn2d, dflat, W, ot)

    BB = 256
    out = pl.pallas_call(
        _tc_dot_kernel,
        grid=(B // BB,),
        in_specs=[
            pl.BlockSpec((BB, V), lambda i: (i, 0)),
            pl.BlockSpec((BB, NP1, V), lambda i: (i, 0, 0)),
        ],
        out_specs=pl.BlockSpec((BB, NP1), lambda i: (i, 0)),
        out_shape=jax.ShapeDtypeStruct((B, NP1), jnp.float32),
    )(x, otg.reshape(B, NP1, V))
    return out


# speedup vs baseline: 1.4398x; 1.0232x over previous
import jax
import jax.numpy as jnp
from jax import lax
from jax.experimental import pallas as pl
from jax.experimental.pallas import tpu as pltpu
from jax.experimental.pallas import tpu_sc as plsc

B, C, NP1 = 4096, 20, 26
ND, NW, V = 100, 10000, 64
L = 16
NC, NS = 2, 16
NWORK = NC * NS
BPW = B // NWORK
IW = 128
NJD = BPW * C // IW
NJO = BPW * NP1 // IW
NB = 3


def _zero_acc(zbuf, zidx_v, xacc, iota, base, sem):
    zero = jnp.zeros((L,), jnp.float32)
    for r in range(IW):
        for s in range(V // L):
            zbuf[r, pl.ds(s * L, L)] = zero
    for i in range(IW // L):
        zidx_v[0, pl.ds(i * L, L)] = iota + (base + i * L)
    return pltpu.async_copy(zbuf, xacc.at[zidx_v.at[0]], sem)


def _fill_tgt(tgt_v, iota, base):
    for i in range(BPW * C // L):
        j, col = i * L // IW, i * L % IW
        bl = lax.div(iota + (i * L), C)
        tgt_v[j, pl.ds(col, L)] = bl + base


def _sc_a_kernel(ctx_hbm, tn_hbm, w_hbm, ot_hbm,
                 xw_hbm, otg_hbm,
                 ctx_v, tn_v, tgt_v, wbuf, obuf, zbuf, zidx_v, xacc,
                 semwg, semwa, semog, semow):
    cid = lax.axis_index("c")
    sid = lax.axis_index("s")
    wid = cid * NS + sid
    iota = lax.iota(jnp.int32, L)
    base = sid * BPW

    zd = _zero_acc(zbuf, zidx_v, xacc, iota, base, semwa)
    pltpu.sync_copy(ctx_hbm.at[pl.ds(wid * NJD, NJD)], ctx_v)
    pltpu.sync_copy(tn_hbm.at[pl.ds(wid * NJO, NJO)], tn_v)
    _fill_tgt(tgt_v, iota, base)
    zd.wait()

    wg = [None] * NJD
    wa = [None] * NJD
    og = [None] * NJO
    ow = [None] * NJO
    for t in range(NJO + 1):
        if t < NJD:
            if t >= NB:
                wa[t - NB].wait()
            wg[t] = pltpu.async_copy(w_hbm.at[ctx_v.at[t]],
                                     wbuf.at[t % NB], semwg)
        if t < NJO:
            if t >= NB:
                ow[t - NB].wait()
            og[t] = pltpu.async_copy(ot_hbm.at[tn_v.at[t]],
                                     obuf.at[t % NB], semog)
        u = t - 1
        if 0 <= u < NJD:
            wg[u].wait()
            wa[u] = pltpu.async_copy(wbuf.at[u % NB], xacc.at[tgt_v.at[u]],
                                     semwa, add=True)
        if 0 <= u < NJO:
            og[u].wait()
            ow[u] = pltpu.async_copy(
                obuf.at[u % NB],
                otg_hbm.at[pl.ds(wid * (BPW * NP1) + u * IW, IW)], semow)

    for u in range(max(NJD - NB, 0), NJD):
        wa[u].wait()
    for u in range(max(NJO - NB, 0), NJO):
        ow[u].wait()
    plsc.subcore_barrier()
    pltpu.sync_copy(xacc.at[pl.ds(base, BPW)],
                    xw_hbm.at[pl.ds(wid * BPW, BPW)])


def _sc_b_kernel(fidx_hbm, dflat_hbm,
                 xd_hbm,
                 didx_v, tgt_v, dbuf, zbuf, zidx_v, xacc, semdg, semda):
    cid = lax.axis_index("c")
    sid = lax.axis_index("s")
    wid = cid * NS + sid
    iota = lax.iota(jnp.int32, L)
    base = sid * BPW

    zd = _zero_acc(zbuf, zidx_v, xacc, iota, base, semda)
    pltpu.sync_copy(fidx_hbm.at[pl.ds(wid * NJD, NJD)], didx_v)
    _fill_tgt(tgt_v, iota, base)
    zd.wait()

    dg = [None] * NJD
    da = [None] * NJD
    for t in range(NJD + 1):
        if t < NJD:
            if t >= NB:
                da[t - NB].wait()
            dg[t] = pltpu.async_copy(dflat_hbm.at[didx_v.at[t]],
                                     dbuf.at[t % NB], semdg)
        u = t - 1
        if 0 <= u < NJD:
            dg[u].wait()
            da[u] = pltpu.async_copy(dbuf.at[u % NB], xacc.at[tgt_v.at[u]],
                                     semda, add=True)
    for u in range(max(NJD - NB, 0), NJD):
        da[u].wait()
    plsc.subcore_barrier()
    pltpu.sync_copy(xacc.at[pl.ds(base, BPW)],
                    xd_hbm.at[pl.ds(wid * BPW, BPW)])


def _tc_transpose_kernel(o_ref, ot_ref):
    ot_ref[...] = o_ref[...].T


def _tc_dot_kernel(xw_ref, xd_ref, og_ref, out_ref):
    x = xw_ref[...] + xd_ref[...]
    og = og_ref[...]
    out_ref[...] = jnp.sum(og * x[:, None, :], axis=-1)


_SC_PARAMS = dict(
    compiler_params=pltpu.CompilerParams(
        needs_layout_passes=False, use_tc_tiling_on_sc=False),
)


def _run_sc_a(ctx2d, tn2d, W, ot):
    mesh = plsc.VectorSubcoreMesh(core_axis_name="c", subcore_axis_name="s")
    sc = pl.kernel(
        _sc_a_kernel,
        out_type=(
            jax.ShapeDtypeStruct((B, V), jnp.float32),
            jax.ShapeDtypeStruct((B * NP1, V), jnp.float32),
        ),
        mesh=mesh,
        scratch_types=[
            pltpu.VMEM((NJD, IW), jnp.int32),
            pltpu.VMEM((NJO, IW), jnp.int32),
            pltpu.VMEM((NJD, IW), jnp.int32),
            pltpu.VMEM((NB, IW, V), jnp.float32),
            pltpu.VMEM((NB, IW, V), jnp.float32),
            pltpu.VMEM((IW, V), jnp.float32),
            pltpu.VMEM((1, IW), jnp.int32),
            pltpu.VMEM_SHARED((NS * BPW, V), jnp.float32),
            pltpu.SemaphoreType.DMA,
            pltpu.SemaphoreType.DMA,
            pltpu.SemaphoreType.DMA,
            pltpu.SemaphoreType.DMA,
        ],
        **_SC_PARAMS,
    )
    return sc(ctx2d, tn2d, W, ot)


def _run_sc_b(fidx2d, dflat):
    mesh = plsc.VectorSubcoreMesh(core_axis_name="c", subcore_axis_name="s")
    sc = pl.kernel(
        _sc_b_kernel,
        out_type=jax.ShapeDtypeStruct((B, V), jnp.float32),
        mesh=mesh,
        scratch_types=[
            pltpu.VMEM((NJD, IW), jnp.int32),
            pltpu.VMEM((NJD, IW), jnp.int32),
            pltpu.VMEM((NB, IW, V), jnp.float32),
            pltpu.VMEM((IW, V), jnp.float32),
            pltpu.VMEM((1, IW), jnp.int32),
            pltpu.VMEM_SHARED((NS * BPW, V), jnp.float32),
            pltpu.SemaphoreType.DMA,
            pltpu.SemaphoreType.DMA,
        ],
        **_SC_PARAMS,
    )
    return sc(fidx2d, dflat)


def kernel(context_ids, doc_ids, target_noise_ids, D, W, O):
    ctx = context_ids.astype(jnp.int32)
    doc = doc_ids.astype(jnp.int32)
    ctx2d = ctx.reshape(B * C // IW, IW)
    fidx2d = (doc[:, None] * NW + ctx).reshape(B * C // IW, IW)
    tn2d = target_noise_ids.astype(jnp.int32).reshape(B * NP1 // IW, IW)
    dflat = D.reshape(ND * NW, V)

    ot = pl.pallas_call(
        _tc_transpose_kernel,
        out_shape=jax.ShapeDtypeStruct((NW, V), jnp.float32),
    )(O)

    xw, otg = _run_sc_a(ctx2d, tn2d, W, ot)
    xd = _run_sc_b(fidx2d, dflat)

    BB = 256
    out = pl.pallas_call(
        _tc_dot_kernel,
        grid=(B // BB,),
        in_specs=[
            pl.BlockSpec((BB, V), lambda i: (i, 0)),
            pl.BlockSpec((BB, V), lambda i: (i, 0)),
            pl.BlockSpec((BB, NP1, V), lambda i: (i, 0, 0)),
        ],
        out_specs=pl.BlockSpec((BB, NP1), lambda i: (i, 0)),
        out_shape=jax.ShapeDtypeStruct((B, NP1), jnp.float32),
    )(xw, xd, otg.reshape(B, NP1, V))
    return out
